# baseline (speedup 1.0000x reference)
import jax
import jax.numpy as jnp
from jax.experimental import pallas as pl
from jax.experimental.pallas import tpu as pltpu

B, S, DV, DA = 64, 512, 1024, 512
DM = 2 * DA

BB = 4
SC = 128
NCHUNK = S // SC
NSTEP = B // BB
M = BB * SC


def _attn_kernel(vec_ref, mat_ref, len_ref, wv_ref, wm_ref, wa_ref,
                 reps_ref, attn_ref, rbf_ref, attn_s_ref):
    f32 = jnp.float32
    j = pl.program_id(0)
    slot = jax.lax.rem(j, 2)
    prev_slot = jax.lax.rem(j + 1, 2)

    @pl.when(j > 0)
    def _band():
        ab = attn_s_ref[prev_slot]
        sub = jax.lax.broadcasted_iota(jnp.int32, (BB, M), 0)
        blk = jax.lax.broadcasted_iota(jnp.int32, (BB, M), 1) // SC
        on_band = sub == blk
        band_cs = []
        for c in range(NCHUNK):
            ac = ab[:, c * SC:(c + 1) * SC]
            band_cs.append(
                jnp.where(on_band, jnp.concatenate([ac] * BB, axis=1), 0.0))
        band = jnp.concatenate(band_cs, axis=1)
        reps_ref[0] = jnp.dot(band.astype(jnp.bfloat16), rbf_ref[prev_slot],
                              preferred_element_type=f32)

    @pl.when(j < NSTEP)
    def _logits():
        t1 = jnp.dot(vec_ref[0], wv_ref[...], preferred_element_type=f32)
        t1big = jnp.broadcast_to(t1[:, None, :], (BB, SC, DA)).reshape(M, DA)
        wa = wa_ref[...]
        logit_chunks = []
        for c in range(NCHUNK):
            rows = mat_ref[0, :, c * SC:(c + 1) * SC, :].reshape(M, DM)
            rbf_ref[slot, c * M:(c + 1) * M, :] = rows.astype(jnp.bfloat16)
            t2 = jnp.dot(rows, wm_ref[...], preferred_element_type=f32)
            t3 = jnp.maximum(t2 + t1big, 0.0) * wa
            logit_chunks.append(jnp.sum(t3.reshape(BB, SC, DA), axis=-1))
        logits = jnp.concatenate(logit_chunks, axis=1)

        rowmax = jnp.max(logits, axis=-1, keepdims=True)
        unnorm = jnp.exp(logits - rowmax)
        seq = jax.lax.broadcasted_iota(jnp.int32, (BB, S), 1)
        masked = jnp.where(seq < len_ref[0], unnorm, 0.0)
        attn = masked / jnp.sum(masked, axis=-1, keepdims=True)
        attn_ref[0] = attn
        attn_s_ref[slot] = attn


def kernel(vector, matrix, input_lengths, W_vec, W_mat, w_attn):
    lengths = input_lengths.astype(jnp.int32).reshape(NSTEP, BB, 1)
    wa2 = w_attn.reshape(1, DA)
    vec4 = vector.reshape(NSTEP, BB, DV)
    mat4 = matrix.reshape(NSTEP, BB, S, DM)

    last = NSTEP - 1
    reps, attn = pl.pallas_call(
        _attn_kernel,
        out_shape=(
            jax.ShapeDtypeStruct((NSTEP, BB, DM), jnp.float32),
            jax.ShapeDtypeStruct((NSTEP, BB, S), jnp.float32),
        ),
        grid=(NSTEP + 1,),
        in_specs=[
            pl.BlockSpec((1, BB, DV), lambda j: (jnp.minimum(j, last), 0, 0)),
            pl.BlockSpec((1, BB, S, DM),
                         lambda j: (jnp.minimum(j, last), 0, 0, 0)),
            pl.BlockSpec((1, BB, 1), lambda j: (jnp.minimum(j, last), 0, 0)),
            pl.BlockSpec((DV, DA), lambda j: (0, 0)),
            pl.BlockSpec((DM, DA), lambda j: (0, 0)),
            pl.BlockSpec((1, DA), lambda j: (0, 0)),
        ],
        out_specs=(
            pl.BlockSpec((1, BB, DM), lambda j: (jnp.maximum(j - 1, 0), 0, 0)),
            pl.BlockSpec((1, BB, S), lambda j: (jnp.minimum(j, last), 0, 0)),
        ),
        scratch_shapes=[
            pltpu.VMEM((2, NCHUNK * M, DM), jnp.bfloat16),
            pltpu.VMEM((2, BB, S), jnp.float32),
        ],
        compiler_params=pltpu.CompilerParams(
            dimension_semantics=("arbitrary",),
            vmem_limit_bytes=50 * 1024 * 1024,
        ),
        name="attention_pair",
    )(vec4, mat4, lengths, W_vec, W_mat, wa2)
    return reps.reshape(B, DM), attn.reshape(B, S)


# speedup vs baseline: 1.0605x; 1.0605x over previous
import jax
import jax.numpy as jnp
from jax.experimental import pallas as pl
from jax.experimental.pallas import tpu as pltpu

B, S, DV, DA = 64, 512, 1024, 512
DM = 2 * DA

BB = 8
SC = 128
NCHUNK = S // SC
NSTEP = B // BB
M = BB * SC


def _attn_kernel(vec_ref, mat_ref, len_ref, wv_ref, wm_ref, wa_ref,
                 reps_ref, attn_ref, rbf_ref, attn_s_ref):
    f32 = jnp.float32
    j = pl.program_id(0)
    slot = jax.lax.rem(j, 2)
    prev_slot = jax.lax.rem(j + 1, 2)

    @pl.when(j > 0)
    def _band():
        ab = attn_s_ref[prev_slot]
        sub = jax.lax.broadcasted_iota(jnp.int32, (BB, M), 0)
        blk = jax.lax.broadcasted_iota(jnp.int32, (BB, M), 1) // SC
        on_band = sub == blk
        band_cs = []
        for c in range(NCHUNK):
            ac = ab[:, c * SC:(c + 1) * SC]
            band_cs.append(
                jnp.where(on_band, jnp.concatenate([ac] * BB, axis=1), 0.0))
        band = jnp.concatenate(band_cs, axis=1)
        reps_ref[...] = jnp.dot(band.astype(jnp.bfloat16), rbf_ref[...],
                                preferred_element_type=f32)

    @pl.when(j < NSTEP)
    def _logits():
        t1 = jnp.dot(vec_ref[...], wv_ref[...], preferred_element_type=f32)
        t1big = jnp.broadcast_to(t1[:, None, :], (BB, SC, DA)).reshape(M, DA)
        wa = wa_ref[...]
        logit_chunks = []
        for c in range(NCHUNK):
            rows = mat_ref[:, c * SC:(c + 1) * SC, :].reshape(M, DM)
            rbf_ref[c * M:(c + 1) * M, :] = rows.astype(jnp.bfloat16)
            t2 = jnp.dot(rows, wm_ref[...], preferred_element_type=f32)
            t3 = jnp.maximum(t2 + t1big, 0.0) * wa
            logit_chunks.append(jnp.sum(t3.reshape(BB, SC, DA), axis=-1))
        logits = jnp.concatenate(logit_chunks, axis=1)

        rowmax = jnp.max(logits, axis=-1, keepdims=True)
        unnorm = jnp.exp(logits - rowmax)
        seq = jax.lax.broadcasted_iota(jnp.int32, (BB, S), 1)
        masked = jnp.where(seq < len_ref[...], unnorm, 0.0)
        attn = masked / jnp.sum(masked, axis=-1, keepdims=True)
        attn_ref[...] = attn
        attn_s_ref[slot] = attn


def kernel(vector, matrix, input_lengths, W_vec, W_mat, w_attn):
    lengths = input_lengths.astype(jnp.int32).reshape(B, 1)
    wa2 = w_attn.reshape(1, DA)

    last = NSTEP - 1
    reps, attn = pl.pallas_call(
        _attn_kernel,
        out_shape=(
            jax.ShapeDtypeStruct((B, DM), jnp.float32),
            jax.ShapeDtypeStruct((B, S), jnp.float32),
        ),
        grid=(NSTEP + 1,),
        in_specs=[
            pl.BlockSpec((BB, DV), lambda j: (jnp.minimum(j, last), 0)),
            pl.BlockSpec((BB, S, DM), lambda j: (jnp.minimum(j, last), 0, 0)),
            pl.BlockSpec((BB, 1), lambda j: (jnp.minimum(j, last), 0)),
            pl.BlockSpec((DV, DA), lambda j: (0, 0)),
            pl.BlockSpec((DM, DA), lambda j: (0, 0)),
            pl.BlockSpec((1, DA), lambda j: (0, 0)),
        ],
        out_specs=(
            pl.BlockSpec((BB, DM), lambda j: (jnp.maximum(j - 1, 0), 0)),
            pl.BlockSpec((BB, S), lambda j: (jnp.minimum(j, last), 0)),
        ),
        scratch_shapes=[
            pltpu.VMEM((NCHUNK * M, DM), jnp.bfloat16),
            pltpu.VMEM((2, BB, S), jnp.float32),
        ],
        compiler_params=pltpu.CompilerParams(
            dimension_semantics=("arbitrary",),
            vmem_limit_bytes=56 * 1024 * 1024,
        ),
        name="attention_pair",
    )(vector, matrix, lengths, W_vec, W_mat, wa2)
    return reps, attn


# speedup vs baseline: 1.1042x; 1.0412x over previous
import jax
import jax.numpy as jnp
from jax.experimental import pallas as pl
from jax.experimental.pallas import tpu as pltpu

B, S, DV, DA = 64, 512, 1024, 512
DM = 2 * DA

BB = 8
SC = 128
NCHUNK = S // SC


def _attn_kernel(vec_ref, mat_ref, mat2_ref, len_ref, wv_ref, wm_ref, wa_ref,
                 reps_ref, attn_ref):
    f32 = jnp.float32
    t1 = jnp.dot(vec_ref[...], wv_ref[...], preferred_element_type=f32)

    t1big = jnp.broadcast_to(t1[:, None, :], (BB, SC, DA)).reshape(BB * SC, DA)
    wa = wa_ref[...]

    logit_chunks = []
    for c in range(NCHUNK):
        ref = mat_ref if c < NCHUNK // 2 else mat2_ref
        cc = c % (NCHUNK // 2)
        rows = ref[:, cc * SC:(cc + 1) * SC, :].reshape(BB * SC, DM)
        t2 = jnp.dot(rows, wm_ref[...], preferred_element_type=f32)
        t3 = jnp.maximum(t2 + t1big, 0.0) * wa
        logit_chunks.append(jnp.sum(t3.reshape(BB, SC, DA), axis=-1))
    logits = jnp.concatenate(logit_chunks, axis=1)

    rowmax = jnp.max(logits, axis=-1, keepdims=True)
    unnorm = jnp.exp(logits - rowmax)
    seq = jax.lax.broadcasted_iota(jnp.int32, (BB, S), 1)
    masked = jnp.where(seq < len_ref[...], unnorm, 0.0)
    denom = jnp.sum(masked, axis=-1, keepdims=True)
    attn = masked / denom
    attn_ref[...] = attn

    H = S // 2
    sub = jax.lax.broadcasted_iota(jnp.int32, (BB, BB * H), 0)
    blk = jax.lax.broadcasted_iota(jnp.int32, (BB, BB * H), 1) // H
    on_band = sub == blk
    b1 = jnp.where(on_band, jnp.concatenate([attn[:, :H]] * BB, axis=1), 0.0)
    b2 = jnp.where(on_band, jnp.concatenate([attn[:, H:]] * BB, axis=1), 0.0)
    r1 = jnp.dot(b1, mat_ref[...].reshape(BB * H, DM),
                 preferred_element_type=f32)
    r2 = jnp.dot(b2, mat2_ref[...].reshape(BB * H, DM),
                 preferred_element_type=f32)
    reps_ref[...] = r1 + r2


def kernel(vector, matrix, input_lengths, W_vec, W_mat, w_attn):
    lengths = input_lengths.astype(jnp.int32).reshape(B, 1)
    wa2 = w_attn.reshape(1, DA)

    grid = (B // BB,)
    reps, attn = pl.pallas_call(
        _attn_kernel,
        out_shape=(
            jax.ShapeDtypeStruct((B, DM), jnp.float32),
            jax.ShapeDtypeStruct((B, S), jnp.float32),
        ),
        grid=grid,
        in_specs=[
            pl.BlockSpec((BB, DV), lambda i: (i, 0)),
            pl.BlockSpec((BB, S // 2, DM), lambda i: (i, 0, 0)),
            pl.BlockSpec((BB, S // 2, DM), lambda i: (i, 1, 0)),
            pl.BlockSpec((BB, 1), lambda i: (i, 0)),
            pl.BlockSpec((DV, DA), lambda i: (0, 0)),
            pl.BlockSpec((DM, DA), lambda i: (0, 0)),
            pl.BlockSpec((1, DA), lambda i: (0, 0)),
        ],
        out_specs=(
            pl.BlockSpec((BB, DM), lambda i: (i, 0)),
            pl.BlockSpec((BB, S), lambda i: (i, 0)),
        ),
        compiler_params=pltpu.CompilerParams(
            dimension_semantics=("arbitrary",),
            vmem_limit_bytes=50 * 1024 * 1024,
        ),
        name="attention_pair",
    )(vector, matrix, matrix, lengths, W_vec, W_mat, wa2)
    return reps, attn
